# Initial kernel scaffold; baseline (speedup 1.0000x reference)
import jax
import jax.numpy as jnp
from jax.experimental import pallas as pl


def kernel(u, Wg, W1s, b1s, W2s, b2s, W1, b1, W2, b2):
    raise NotImplementedError("write your pallas kernel here")


# speedup vs baseline: 1.1183x; 1.1183x over previous
import functools

import jax
import jax.numpy as jnp
from jax.experimental import pallas as pl
from jax.experimental.pallas import tpu as pltpu


def _moe_step(u_ref, wg_ref, w1_ref, b1_ref, w2_ref, b2_ref, out_ref,
              comb_ref, ub_ref, *, n_shared, n_routed):
    j = pl.program_id(0)

    @pl.when(j == 0)
    def _router():
        u = u_ref[...]
        ub_ref[...] = u.astype(jnp.bfloat16)
        logits = jax.lax.dot_general(
            u, wg_ref[...], (((1,), (0,)), ((), ())),
            preferred_element_type=jnp.float32,
            precision=jax.lax.Precision.HIGHEST)
        m = jnp.min(logits, axis=-1, keepdims=True)
        idx = jax.lax.broadcasted_iota(jnp.int32, logits.shape, 1)
        excl = jnp.max(jnp.where(logits <= m, idx, -1), axis=-1,
                       keepdims=True)
        keep = idx != excl
        mx = jnp.max(logits, axis=-1, keepdims=True)
        ex = jnp.where(keep, jnp.exp(logits - mx), 0.0)
        comb_ref[...] = ex / jnp.sum(ex, axis=-1, keepdims=True)

    ub = ub_ref[...]
    h = jax.lax.dot_general(
        ub, w1_ref[0], (((1,), (0,)), ((), ())),
        preferred_element_type=jnp.float32)
    h = jnp.maximum(h + b1_ref[0], 0.0).astype(jnp.bfloat16)
    o = jax.lax.dot_general(
        h, w2_ref[0], (((1,), (0,)), ((), ())),
        preferred_element_type=jnp.float32)
    o = o + b2_ref[0]

    @pl.when(j == 0)
    def _init():
        out_ref[...] = u_ref[...] + o * (1.0 / n_shared)

    @pl.when(jnp.logical_and(j > 0, j < n_shared))
    def _shared():
        out_ref[...] = out_ref[...] + o * (1.0 / n_shared)

    @pl.when(j >= n_shared)
    def _routed():
        e = j - n_shared
        comb = comb_ref[...]
        lane = jax.lax.broadcasted_iota(jnp.int32, comb.shape, 1)
        w = jnp.sum(jnp.where(lane == e, comb, 0.0), axis=1, keepdims=True)
        out_ref[...] = out_ref[...] + w * o


@functools.partial(jax.jit, static_argnames=())
def kernel(u, Wg, W1s, b1s, W2s, b2s, W1, b1, W2, b2):
    T, Dm = u.shape
    n_shared = W1s.shape[0]
    n_routed = W1.shape[0]
    steps = n_shared + n_routed
    w1cat = jnp.concatenate([W1s, W1], axis=0).astype(jnp.bfloat16)
    w2cat = jnp.concatenate([W2s, W2], axis=0).astype(jnp.bfloat16)
    b1cat = jnp.concatenate([b1s, b1], axis=0)[:, None, :]
    b2cat = jnp.concatenate([b2s, b2], axis=0)[:, None, :]
    Hh = w1cat.shape[-1]

    grid = (steps,)
    out = pl.pallas_call(
        functools.partial(_moe_step, n_shared=n_shared, n_routed=n_routed),
        grid=grid,
        in_specs=[
            pl.BlockSpec((T, Dm), lambda j: (0, 0)),
            pl.BlockSpec(Wg.shape, lambda j: (0, 0)),
            pl.BlockSpec((1, Dm, Hh), lambda j: (j, 0, 0)),
            pl.BlockSpec((1, 1, Hh), lambda j: (j, 0, 0)),
            pl.BlockSpec((1, Hh, Dm), lambda j: (j, 0, 0)),
            pl.BlockSpec((1, 1, Dm), lambda j: (j, 0, 0)),
        ],
        out_specs=pl.BlockSpec((T, Dm), lambda j: (0, 0)),
        out_shape=jax.ShapeDtypeStruct((T, Dm), jnp.float32),
        scratch_shapes=[
            pltpu.VMEM((T, n_routed), jnp.float32),
            pltpu.VMEM((T, Dm), jnp.bfloat16),
        ],
    )(u, Wg, w1cat, b1cat, w2cat, b2cat)
    return out


# speedup vs baseline: 1.1682x; 1.0446x over previous
import functools

import jax
import jax.numpy as jnp
from jax.experimental import pallas as pl
from jax.experimental.pallas import tpu as pltpu


def _moe_step(u_ref, wg_ref, w1_ref, b1_ref, w2_ref, b2_ref, out_ref,
              comb_ref, ub_ref, *, n_shared, n_routed):
    j = pl.program_id(0)
    steps = n_shared + n_routed

    @pl.when(j == 0)
    def _router():
        u = u_ref[...]
        ub_ref[...] = u.astype(jnp.bfloat16)
        logits = jax.lax.dot_general(
            u, wg_ref[...], (((1,), (0,)), ((), ())),
            preferred_element_type=jnp.float32)
        m = jnp.min(logits, axis=-1, keepdims=True)
        idx = jax.lax.broadcasted_iota(jnp.int32, logits.shape, 1)
        excl = jnp.max(jnp.where(logits <= m, idx, -1), axis=-1,
                       keepdims=True)
        keep = idx != excl
        mx = jnp.max(logits, axis=-1, keepdims=True)
        ex = jnp.where(keep, jnp.exp(logits - mx), 0.0)
        sm = ex / jnp.sum(ex, axis=-1, keepdims=True)
        lane9 = jax.lax.broadcasted_iota(jnp.int32, (u.shape[0], 16), 1)
        shared_w = jnp.where(lane9 < n_shared, 1.0 / n_shared, 0.0)
        routed_w = jnp.where(
            jnp.logical_and(lane9 >= n_shared, lane9 < steps),
            jnp.pad(sm, ((0, 0), (n_shared, 16 - steps))), 0.0)
        comb_ref[...] = shared_w + routed_w

    comb = comb_ref[...]
    lane = jax.lax.broadcasted_iota(jnp.int32, comb.shape, 1)
    w = jnp.sum(jnp.where(lane == j, comb, 0.0), axis=1, keepdims=True)

    ub = ub_ref[...]
    h = jax.lax.dot_general(
        ub, w1_ref[0], (((1,), (0,)), ((), ())),
        preferred_element_type=jnp.float32)
    h = (jnp.maximum(h + b1_ref[0], 0.0) * w).astype(jnp.bfloat16)
    o = jax.lax.dot_general(
        h, w2_ref[0], (((1,), (0,)), ((), ())),
        preferred_element_type=jnp.float32)

    @pl.when(j == 0)
    def _init():
        out_ref[...] = o

    @pl.when(jnp.logical_and(j > 0, j < steps - 1))
    def _acc():
        out_ref[...] = out_ref[...] + o

    @pl.when(j == steps - 1)
    def _fin():
        bias = jax.lax.dot_general(
            comb, b2_ref[...], (((1,), (0,)), ((), ())),
            preferred_element_type=jnp.float32)
        out_ref[...] = out_ref[...] + o + u_ref[...] + bias


@functools.partial(jax.jit, static_argnames=())
def kernel(u, Wg, W1s, b1s, W2s, b2s, W1, b1, W2, b2):
    T, Dm = u.shape
    n_shared = W1s.shape[0]
    n_routed = W1.shape[0]
    steps = n_shared + n_routed
    w1cat = jnp.concatenate([W1s, W1], axis=0).astype(jnp.bfloat16)
    w2cat = jnp.concatenate([W2s, W2], axis=0).astype(jnp.bfloat16)
    b1cat = jnp.concatenate([b1s, b1], axis=0)[:, None, :]
    b2cat = jnp.pad(jnp.concatenate([b2s, b2], axis=0),
                    ((0, 16 - steps), (0, 0)))
    Hh = w1cat.shape[-1]

    grid = (steps,)
    out = pl.pallas_call(
        functools.partial(_moe_step, n_shared=n_shared, n_routed=n_routed),
        grid=grid,
        in_specs=[
            pl.BlockSpec((T, Dm), lambda j: (0, 0)),
            pl.BlockSpec(Wg.shape, lambda j: (0, 0)),
            pl.BlockSpec((1, Dm, Hh), lambda j: (j, 0, 0)),
            pl.BlockSpec((1, 1, Hh), lambda j: (j, 0, 0)),
            pl.BlockSpec((1, Hh, Dm), lambda j: (j, 0, 0)),
            pl.BlockSpec((16, Dm), lambda j: (0, 0)),
        ],
        out_specs=pl.BlockSpec((T, Dm), lambda j: (0, 0)),
        out_shape=jax.ShapeDtypeStruct((T, Dm), jnp.float32),
        scratch_shapes=[
            pltpu.VMEM((T, 16), jnp.float32),
            pltpu.VMEM((T, Dm), jnp.bfloat16),
        ],
    )(u, Wg, w1cat, b1cat, w2cat, b2cat)
    return out


# speedup vs baseline: 1.3972x; 1.1961x over previous
import functools

import jax
import jax.numpy as jnp
from jax.experimental import pallas as pl
from jax.experimental.pallas import tpu as pltpu


def _moe_step(u_ref, wg_ref, w1s_ref, b1s_ref, w2s_ref, w1_ref, b1_ref,
              w2_ref, b2all_ref, out_ref, comb_ref, *, n_shared, n_routed):
    j = pl.program_id(0)
    steps = n_shared + n_routed

    @pl.when(j == 0)
    def _router():
        u = u_ref[...]
        logits = jax.lax.dot_general(
            u, wg_ref[...], (((1,), (0,)), ((), ())),
            preferred_element_type=jnp.float32)
        m = jnp.min(logits, axis=-1, keepdims=True)
        idx = jax.lax.broadcasted_iota(jnp.int32, logits.shape, 1)
        excl = jnp.max(jnp.where(logits <= m, idx, -1), axis=-1,
                       keepdims=True)
        keep = idx != excl
        mx = jnp.max(logits, axis=-1, keepdims=True)
        ex = jnp.where(keep, jnp.exp(logits - mx), 0.0)
        sm = ex / jnp.sum(ex, axis=-1, keepdims=True)
        lane = jax.lax.broadcasted_iota(jnp.int32, (u.shape[0], 16), 1)
        shared_w = jnp.where(lane < n_shared, 1.0 / n_shared, 0.0)
        routed_w = jnp.where(
            jnp.logical_and(lane >= n_shared, lane < steps),
            jnp.pad(sm, ((0, 0), (n_shared, 16 - steps))), 0.0)
        comb_ref[...] = shared_w + routed_w

    comb = comb_ref[...]
    lane = jax.lax.broadcasted_iota(jnp.int32, comb.shape, 1)
    w = jnp.sum(jnp.where(lane == j, comb, 0.0), axis=1, keepdims=True)

    shared_step = j < n_shared
    u = u_ref[...]

    def ffn(w1, b1, w2):
        h = jax.lax.dot_general(
            u, w1, (((1,), (0,)), ((), ())),
            preferred_element_type=jnp.float32)
        h = jnp.maximum(h + b1, 0.0) * w
        return jax.lax.dot_general(
            h, w2, (((1,), (0,)), ((), ())),
            preferred_element_type=jnp.float32)

    @pl.when(shared_step)
    def _shared():
        o = ffn(w1s_ref[0], b1s_ref[0], w2s_ref[0])

        @pl.when(j == 0)
        def _init():
            out_ref[...] = o

        @pl.when(j > 0)
        def _acc():
            out_ref[...] = out_ref[...] + o

    @pl.when(jnp.logical_not(shared_step))
    def _routed():
        o = ffn(w1_ref[0], b1_ref[0], w2_ref[0])

        @pl.when(j < steps - 1)
        def _acc():
            out_ref[...] = out_ref[...] + o

        @pl.when(j == steps - 1)
        def _fin():
            bias = jax.lax.dot_general(
                comb, b2all_ref[...], (((1,), (0,)), ((), ())),
                preferred_element_type=jnp.float32)
            out_ref[...] = out_ref[...] + o + u + bias


@functools.partial(jax.jit, static_argnames=())
def kernel(u, Wg, W1s, b1s, W2s, b2s, W1, b1, W2, b2):
    T, Dm = u.shape
    n_shared = W1s.shape[0]
    n_routed = W1.shape[0]
    steps = n_shared + n_routed
    Hh = W1.shape[-1]
    b1s3 = b1s[:, None, :]
    b13 = b1[:, None, :]
    b2all = jnp.pad(jnp.concatenate([b2s, b2], axis=0),
                    ((0, 16 - steps), (0, 0)))

    def ridx(j):
        return (jnp.maximum(j, n_shared) - n_shared, 0, 0)

    grid = (steps,)
    out = pl.pallas_call(
        functools.partial(_moe_step, n_shared=n_shared, n_routed=n_routed),
        grid=grid,
        in_specs=[
            pl.BlockSpec((T, Dm), lambda j: (0, 0)),
            pl.BlockSpec(Wg.shape, lambda j: (0, 0)),
            pl.BlockSpec((1, Dm, Hh), lambda j: (0, 0, 0)),
            pl.BlockSpec((1, 1, Hh), lambda j: (0, 0, 0)),
            pl.BlockSpec((1, Hh, Dm), lambda j: (0, 0, 0)),
            pl.BlockSpec((1, Dm, Hh), ridx),
            pl.BlockSpec((1, 1, Hh), ridx),
            pl.BlockSpec((1, Hh, Dm), ridx),
            pl.BlockSpec((16, Dm), lambda j: (0, 0)),
        ],
        out_specs=pl.BlockSpec((T, Dm), lambda j: (0, 0)),
        out_shape=jax.ShapeDtypeStruct((T, Dm), jnp.float32),
        scratch_shapes=[
            pltpu.VMEM((T, 16), jnp.float32),
        ],
    )(u, Wg, W1s, b1s3, W2s, W1, b13, W2, b2all)
    return out


# speedup vs baseline: 1.4938x; 1.0691x over previous
import functools

import jax
import jax.numpy as jnp
from jax.experimental import pallas as pl
from jax.experimental.pallas import tpu as pltpu


def _moe_tile(u_ref, wg_ref, w1s_ref, b1s_ref, w2s_ref, w1_ref, b1_ref,
              w2r_ref, b2all_ref, out_ref, h_ref, *, n_shared, n_routed):
    steps = n_shared + n_routed
    u = u_ref[...]
    tt = u.shape[0]
    hh = w1s_ref.shape[-1]

    logits = jax.lax.dot_general(
        u, wg_ref[...], (((1,), (0,)), ((), ())),
        preferred_element_type=jnp.float32)
    m = jnp.min(logits, axis=-1, keepdims=True)
    idx = jax.lax.broadcasted_iota(jnp.int32, logits.shape, 1)
    excl = jnp.max(jnp.where(logits <= m, idx, -1), axis=-1, keepdims=True)
    keep = idx != excl
    mx = jnp.max(logits, axis=-1, keepdims=True)
    ex = jnp.where(keep, jnp.exp(logits - mx), 0.0)
    sm = ex / jnp.sum(ex, axis=-1, keepdims=True)
    lane = jax.lax.broadcasted_iota(jnp.int32, (tt, 16), 1)
    shared_w = jnp.where(lane < n_shared, 1.0 / n_shared, 0.0)
    routed_w = jnp.where(
        jnp.logical_and(lane >= n_shared, lane < steps),
        jnp.pad(sm, ((0, 0), (n_shared, 16 - steps))), 0.0)
    comb = shared_w + routed_w

    for s in range(steps):
        if s < n_shared:
            w1 = w1s_ref[s]
            b1 = b1s_ref[s]
        else:
            w1 = w1_ref[s - n_shared]
            b1 = b1_ref[s - n_shared]
        h = jax.lax.dot_general(
            u, w1, (((1,), (0,)), ((), ())),
            preferred_element_type=jnp.float32)
        w = comb[:, s:s + 1]
        h_ref[:, s * hh:(s + 1) * hh] = jnp.maximum(h + b1, 0.0) * w

    hs = h_ref[:, :n_shared * hh]
    hr = h_ref[:, n_shared * hh:]
    o = jax.lax.dot_general(
        hs, w2s_ref[0], (((1,), (0,)), ((), ())),
        preferred_element_type=jnp.float32)
    o = o + jax.lax.dot_general(
        hr, w2r_ref[...], (((1,), (0,)), ((), ())),
        preferred_element_type=jnp.float32)
    bias = jax.lax.dot_general(
        comb, b2all_ref[...], (((1,), (0,)), ((), ())),
        preferred_element_type=jnp.float32)
    out_ref[...] = o + u + bias


@functools.partial(jax.jit, static_argnames=())
def kernel(u, Wg, W1s, b1s, W2s, b2s, W1, b1, W2, b2):
    T, Dm = u.shape
    n_shared = W1s.shape[0]
    n_routed = W1.shape[0]
    steps = n_shared + n_routed
    Hh = W1.shape[-1]
    TT = 256
    w2r = W2.reshape(n_routed * Hh, Dm)
    b1s3 = b1s[:, None, :]
    b13 = b1[:, None, :]
    b2all = jnp.pad(jnp.concatenate([b2s, b2], axis=0),
                    ((0, 16 - steps), (0, 0)))

    grid = (T // TT,)
    out = pl.pallas_call(
        functools.partial(_moe_tile, n_shared=n_shared, n_routed=n_routed),
        grid=grid,
        in_specs=[
            pl.BlockSpec((TT, Dm), lambda i: (i, 0)),
            pl.BlockSpec(Wg.shape, lambda i: (0, 0)),
            pl.BlockSpec(W1s.shape, lambda i: (0, 0, 0)),
            pl.BlockSpec((n_shared, 1, Hh), lambda i: (0, 0, 0)),
            pl.BlockSpec(W2s.shape, lambda i: (0, 0, 0)),
            pl.BlockSpec(W1.shape, lambda i: (0, 0, 0)),
            pl.BlockSpec((n_routed, 1, Hh), lambda i: (0, 0, 0)),
            pl.BlockSpec((n_routed * Hh, Dm), lambda i: (0, 0)),
            pl.BlockSpec((16, Dm), lambda i: (0, 0)),
        ],
        out_specs=pl.BlockSpec((TT, Dm), lambda i: (i, 0)),
        out_shape=jax.ShapeDtypeStruct((T, Dm), jnp.float32),
        scratch_shapes=[
            pltpu.VMEM((TT, steps * Hh), jnp.float32),
        ],
    )(u, Wg, W1s, b1s3, W2s, W1, b13, w2r, b2all)
    return out
